# baseline (speedup 1.0000x reference)
import functools

import jax
import jax.numpy as jnp
from jax import lax
from jax.experimental import pallas as pl
from jax.experimental.pallas import tpu as pltpu
from jax.experimental.pallas import tpu_sc as plsc

B, C, N, P = 8, 192, 576, 768
_BIG = 3.0e38

NW = 32
ROWS_W = (B * N) // NW
SEG = ROWS_W * N


def _tc_body(xp_ref, w_ref, b_ref, y_ref, idx_ref):
    xb = xp_ref[0]
    w = w_ref[...]
    bias = b_ref[...]
    yb = jnp.dot(xb, w, preferred_element_type=jnp.float32) + bias
    y_ref[0] = yb
    g = lax.dot_general(yb, yb, (((1,), (1,)), ((), ())),
                        preferred_element_type=jnp.float32)
    sq = jnp.sum(yb * yb, axis=1)
    work = jnp.maximum(sq[:, None] + sq[None, :] - 2.0 * g, 0.0)
    col = lax.broadcasted_iota(jnp.int32, (N, N), 1)
    lane = lax.broadcasted_iota(jnp.int32, (N, 16), 1)
    idxs = jnp.zeros((N, 16), jnp.int32)
    for k in range(7):
        m = jnp.min(work, axis=1, keepdims=True)
        cand = jnp.where(work == m, col, jnp.int32(1 << 30))
        idx = jnp.min(cand, axis=1, keepdims=True)
        sel = cand == idx
        idxs = jnp.where(lane == k, idx, idxs)
        work = jnp.where(sel, _BIG, work)
    idx_ref[0] = idxs


def _sc_adj_body(idx_hbm, adj_hbm, idx_v, buf):
    wid = lax.axis_index("s") * 2 + lax.axis_index("c")
    pltpu.sync_copy(idx_hbm.at[pl.ds(wid * ROWS_W * 16, ROWS_W * 16)], idx_v)
    il = lax.broadcasted_iota(jnp.int32, (16,), 0)
    zeros = jnp.zeros((16,), jnp.float32)
    ones = jnp.ones((16,), jnp.float32)
    mask7 = il < 7

    def zero_body(i, _):
        buf[pl.ds(i * 16, 16)] = zeros
        return 0

    lax.fori_loop(0, SEG // 16, zero_body, 0)

    def row_body(r, _):
        rowidx = idx_v[pl.ds(r * 16, 16)]
        plsc.store_scatter(buf, [r * N + rowidx], ones, mask=mask7)
        return 0

    lax.fori_loop(0, ROWS_W, row_body, 0)
    pltpu.sync_copy(buf, adj_hbm.at[pl.ds(wid * SEG, SEG)])


@functools.partial(
    pl.kernel,
    out_type=jax.ShapeDtypeStruct((B * N * N,), jnp.float32),
    mesh=plsc.VectorSubcoreMesh(core_axis_name="c", subcore_axis_name="s"),
    compiler_params=pltpu.CompilerParams(needs_layout_passes=False),
    scratch_types=[
        pltpu.VMEM((ROWS_W * 16,), jnp.int32),
        pltpu.VMEM((SEG,), jnp.float32),
    ],
)
def _sc_adj(idx_hbm, adj_hbm, idx_v, buf):
    _sc_adj_body(idx_hbm, adj_hbm, idx_v, buf)


def kernel(x, W_conv, b_conv):
    xp = (x.reshape(B, 3, 24, 16, 24, 16)
            .transpose(0, 2, 4, 1, 3, 5)
            .reshape(B, N, P))
    wm = W_conv.reshape(C, P).T
    bias = b_conv.reshape(1, C)
    y, idxs = pl.pallas_call(
        _tc_body,
        grid=(B,),
        in_specs=[
            pl.BlockSpec((1, N, P), lambda b: (b, 0, 0)),
            pl.BlockSpec((P, C), lambda b: (0, 0)),
            pl.BlockSpec((1, C), lambda b: (0, 0)),
        ],
        out_specs=[
            pl.BlockSpec((1, N, C), lambda b: (b, 0, 0)),
            pl.BlockSpec((1, N, 16), lambda b: (b, 0, 0)),
        ],
        out_shape=[
            jax.ShapeDtypeStruct((B, N, C), jnp.float32),
            jax.ShapeDtypeStruct((B, N, 16), jnp.int32),
        ],
    )(xp, wm, bias)
    adj = _sc_adj(idxs.reshape(B * N * 16)).reshape(B, N, N)
    return (y, adj)


# speedup vs baseline: 1.1248x; 1.1248x over previous
import functools

import jax
import jax.numpy as jnp
from jax import lax
from jax.experimental import pallas as pl
from jax.experimental.pallas import tpu as pltpu
from jax.experimental.pallas import tpu_sc as plsc

B, C, N, P = 8, 192, 576, 768
_BIG = 3.0e38

NW = 32
ROWS_W = (B * N) // NW
SEG = ROWS_W * N


def _tc_body(xp_ref, w_ref, b_ref, y_ref, idx_ref):
    xb = xp_ref[0]
    w = w_ref[...]
    bias = b_ref[...]
    yb = jnp.dot(xb, w, preferred_element_type=jnp.float32) + bias
    y_ref[0] = yb
    g = lax.dot_general(yb, yb, (((1,), (1,)), ((), ())),
                        preferred_element_type=jnp.float32)
    sq = jnp.sum(yb * yb, axis=1)
    work = jnp.maximum(sq[:, None] + sq[None, :] - 2.0 * g, 0.0)
    col = lax.broadcasted_iota(jnp.int32, (N, N), 1)
    lane = lax.broadcasted_iota(jnp.int32, (N, 16), 1)
    idxs = jnp.zeros((N, 16), jnp.int32)
    for k in range(7):
        m = jnp.min(work, axis=1, keepdims=True)
        cand = jnp.where(work == m, col, jnp.int32(1 << 30))
        idx = jnp.min(cand, axis=1, keepdims=True)
        sel = cand == idx
        idxs = jnp.where(lane == k, idx, idxs)
        work = jnp.where(sel, _BIG, work)
    idx_ref[0] = idxs


def _sc_adj_body(idx_hbm, adj_hbm, idx_v, buf):
    wid = lax.axis_index("s") * 2 + lax.axis_index("c")
    b = wid // 4
    r0 = (wid % 4) * ROWS_W
    pltpu.sync_copy(idx_hbm.at[b, pl.ds(r0, ROWS_W)], idx_v)
    il = lax.broadcasted_iota(jnp.int32, (16,), 0)
    zeros = jnp.zeros((16,), jnp.float32)
    ones = jnp.ones((16,), jnp.float32)
    mask7 = il < 7

    def zero_body(r, _):
        for c in range(N // 16):
            buf[r, pl.ds(c * 16, 16)] = zeros
        return 0

    lax.fori_loop(0, ROWS_W, zero_body, 0)

    def row_body(r, _):
        rowidx = idx_v[r]
        rvec = jnp.full((16,), 0, jnp.int32) + r
        plsc.store_scatter(buf, [rvec, rowidx], ones, mask=mask7)
        return 0

    lax.fori_loop(0, ROWS_W, row_body, 0)
    pltpu.sync_copy(buf, adj_hbm.at[b, pl.ds(r0, ROWS_W)])


@functools.partial(
    pl.kernel,
    out_type=jax.ShapeDtypeStruct((B, N, N), jnp.float32),
    mesh=plsc.VectorSubcoreMesh(core_axis_name="c", subcore_axis_name="s"),
    compiler_params=pltpu.CompilerParams(needs_layout_passes=False),
    scratch_types=[
        pltpu.VMEM((ROWS_W, 16), jnp.int32),
        pltpu.VMEM((ROWS_W, N), jnp.float32),
    ],
)
def _sc_adj(idx_hbm, adj_hbm, idx_v, buf):
    _sc_adj_body(idx_hbm, adj_hbm, idx_v, buf)


def kernel(x, W_conv, b_conv):
    xp = (x.reshape(B, 3, 24, 16, 24, 16)
            .transpose(0, 2, 4, 1, 3, 5)
            .reshape(B, N, P))
    wm = W_conv.reshape(C, P).T
    bias = b_conv.reshape(1, C)
    y, idxs = pl.pallas_call(
        _tc_body,
        grid=(B,),
        in_specs=[
            pl.BlockSpec((1, N, P), lambda b: (b, 0, 0)),
            pl.BlockSpec((P, C), lambda b: (0, 0)),
            pl.BlockSpec((1, C), lambda b: (0, 0)),
        ],
        out_specs=[
            pl.BlockSpec((1, N, C), lambda b: (b, 0, 0)),
            pl.BlockSpec((1, N, 16), lambda b: (b, 0, 0)),
        ],
        out_shape=[
            jax.ShapeDtypeStruct((B, N, C), jnp.float32),
            jax.ShapeDtypeStruct((B, N, 16), jnp.int32),
        ],
    )(xp, wm, bias)
    adj = _sc_adj(idxs)
    return (y, adj)


# speedup vs baseline: 2.6990x; 2.3995x over previous
import functools

import jax
import jax.numpy as jnp
from jax import lax
from jax.experimental import pallas as pl
from jax.experimental.pallas import tpu as pltpu
from jax.experimental.pallas import tpu_sc as plsc

B, C, N, P = 8, 192, 576, 768
_BIG = 3.0e38

NW = 32
ROWS_W = (B * N) // NW
SEG = ROWS_W * N


def _tc_body(xp_ref, w_ref, b_ref, y_ref, idx_ref):
    xb = xp_ref[0]
    w = w_ref[...]
    bias = b_ref[...]
    yb = jnp.dot(xb, w, preferred_element_type=jnp.float32) + bias
    y_ref[0] = yb
    g = lax.dot_general(yb, yb, (((1,), (1,)), ((), ())),
                        preferred_element_type=jnp.float32)
    sq = jnp.sum(yb * yb, axis=1)
    work = jnp.maximum(sq[:, None] + sq[None, :] - 2.0 * g, 0.0)
    col = lax.broadcasted_iota(jnp.int32, (N, N), 1)
    lane = lax.broadcasted_iota(jnp.int32, (N, 16), 1)
    idxs = jnp.zeros((N, 16), jnp.int32)
    for k in range(7):
        m = jnp.min(work, axis=1, keepdims=True)
        cand = jnp.where(work == m, col, jnp.int32(1 << 30))
        idx = jnp.min(cand, axis=1, keepdims=True)
        sel = cand == idx
        idxs = jnp.where(lane == k, idx, idxs)
        work = jnp.where(sel, _BIG, work)
    idx_ref[0] = idxs


def _sc_adj_body(idx_hbm, adj_hbm, idx_v, buf):
    wid = lax.axis_index("s") * 2 + lax.axis_index("c")
    b = wid // 4
    r0 = (wid % 4) * ROWS_W
    pltpu.sync_copy(idx_hbm.at[b, pl.ds(r0, ROWS_W)], idx_v)
    il = lax.broadcasted_iota(jnp.int32, (16,), 0)
    zeros = jnp.zeros((16,), jnp.float32)
    ones = jnp.ones((16,), jnp.float32)
    mask7 = il < 7

    def zero_body(r, _):
        for c in range(N // 16):
            buf[r, pl.ds(c * 16, 16)] = zeros
        return 0

    lax.fori_loop(0, ROWS_W, zero_body, 0)

    def row_body(r, _):
        rowidx = idx_v[r]
        rvec = jnp.full((16,), 0, jnp.int32) + r
        plsc.store_scatter(buf, [rvec, rowidx], ones, mask=mask7)
        return 0

    lax.fori_loop(0, ROWS_W, row_body, 0)
    pltpu.sync_copy(buf, adj_hbm.at[b, pl.ds(r0, ROWS_W)])


@functools.partial(
    pl.kernel,
    out_type=jax.ShapeDtypeStruct((B, N, N), jnp.float32),
    mesh=plsc.VectorSubcoreMesh(core_axis_name="c", subcore_axis_name="s"),
    compiler_params=pltpu.CompilerParams(needs_layout_passes=False),
    scratch_types=[
        pltpu.VMEM((ROWS_W, 16), jnp.int32),
        pltpu.VMEM((ROWS_W, N), jnp.float32),
    ],
)
def _sc_adj(idx_hbm, adj_hbm, idx_v, buf):
    _sc_adj_body(idx_hbm, adj_hbm, idx_v, buf)


ROWS_I = (B * 3 * 384) // NW
COMBO_W = ROWS_I // 16


def _sc_im2col_body(x_hbm, xp_hbm, slab, xp_loc, sem):
    wid = lax.axis_index("s") * 2 + lax.axis_index("c")
    b = wid // 4
    ibase = (wid % 4) * 6
    for hh in range(2):
        i0 = ibase + hh * 3
        for c in range(3):
            pltpu.sync_copy(
                x_hbm.at[pl.ds((b * 3 + c) * 384 + i0 * 16, 48)],
                slab.at[pl.ds(c * 48, 48)])
        for c in range(3):

            def move(t, _, c=c):
                u = t // 3
                ii = t % 3
                col = (c * 16 + u) * 16
                src_row = c * 48 + ii * 16 + u
                dst_row = ii * 24
                for j in range(24):
                    xp_loc[dst_row + j, pl.ds(col, 16)] = (
                        slab[src_row, pl.ds(j * 16, 16)])
                return 0

            lax.fori_loop(0, 48, move, 0)
        pltpu.sync_copy(xp_loc, xp_hbm.at[b, pl.ds(i0 * 24, 72)])


@functools.partial(
    pl.kernel,
    out_type=jax.ShapeDtypeStruct((B, N, P), jnp.float32),
    mesh=plsc.VectorSubcoreMesh(core_axis_name="c", subcore_axis_name="s"),
    compiler_params=pltpu.CompilerParams(needs_layout_passes=False),
    scratch_types=[
        pltpu.VMEM((144, 384), jnp.float32),
        pltpu.VMEM((72, P), jnp.float32),
        pltpu.SemaphoreType.DMA,
    ],
)
def _sc_im2col(x_hbm, xp_hbm, slab, xp_loc, sem):
    _sc_im2col_body(x_hbm, xp_hbm, slab, xp_loc, sem)


def kernel(x, W_conv, b_conv):
    xp = _sc_im2col(x.reshape(B * 3 * 384, 384))
    wm = W_conv.reshape(C, P).T
    bias = b_conv.reshape(1, C)
    y, idxs = pl.pallas_call(
        _tc_body,
        grid=(B,),
        in_specs=[
            pl.BlockSpec((1, N, P), lambda b: (b, 0, 0)),
            pl.BlockSpec((P, C), lambda b: (0, 0)),
            pl.BlockSpec((1, C), lambda b: (0, 0)),
        ],
        out_specs=[
            pl.BlockSpec((1, N, C), lambda b: (b, 0, 0)),
            pl.BlockSpec((1, N, 16), lambda b: (b, 0, 0)),
        ],
        out_shape=[
            jax.ShapeDtypeStruct((B, N, C), jnp.float32),
            jax.ShapeDtypeStruct((B, N, 16), jnp.int32),
        ],
    )(xp, wm, bias)
    adj = _sc_adj(idxs)
    return (y, adj)


# speedup vs baseline: 3.2338x; 1.1981x over previous
import functools

import jax
import jax.numpy as jnp
from jax import lax
from jax.experimental import pallas as pl
from jax.experimental.pallas import tpu as pltpu
from jax.experimental.pallas import tpu_sc as plsc

B, C, N, P = 8, 192, 576, 768
_BIG = 3.0e38

NW = 32
ROWS_W = (B * N) // NW
SEG = ROWS_W * N


def _tc_body(xp_ref, w_ref, b_ref, y_ref, idx_ref):
    xb = xp_ref[0]
    w = w_ref[...]
    bias = b_ref[...]
    yb = jnp.dot(xb, w, preferred_element_type=jnp.float32) + bias
    y_ref[0] = yb
    g = lax.dot_general(yb, yb, (((1,), (1,)), ((), ())),
                        preferred_element_type=jnp.float32)
    sq = jnp.sum(yb * yb, axis=1)
    work = jnp.maximum(sq[:, None] + sq[None, :] - 2.0 * g, 0.0)
    col = lax.broadcasted_iota(jnp.int32, (N, N), 1)
    lane = lax.broadcasted_iota(jnp.int32, (N, 16), 1)
    idxs = jnp.zeros((N, 16), jnp.int32)
    for k in range(7):
        m = jnp.min(work, axis=1, keepdims=True)
        cand = jnp.where(work == m, col, jnp.int32(1 << 30))
        idx = jnp.min(cand, axis=1, keepdims=True)
        sel = cand == idx
        idxs = jnp.where(lane == k, idx, idxs)
        work = jnp.where(sel, _BIG, work)
    idx_ref[0] = idxs


def _sc_adj_body(idx_hbm, adj_hbm, idx_v, buf):
    wid = lax.axis_index("s") * 2 + lax.axis_index("c")
    b = wid // 4
    r0 = (wid % 4) * ROWS_W
    pltpu.sync_copy(idx_hbm.at[b, pl.ds(r0, ROWS_W)], idx_v)
    il = lax.broadcasted_iota(jnp.int32, (16,), 0)
    zeros = jnp.zeros((16,), jnp.float32)
    ones = jnp.ones((16,), jnp.float32)
    mask7 = il < 7

    def zero_body(r, _):
        for c in range(N // 16):
            buf[r, pl.ds(c * 16, 16)] = zeros
        return 0

    lax.fori_loop(0, ROWS_W, zero_body, 0)

    def row_body(r, _):
        rowidx = idx_v[r]
        rvec = jnp.full((16,), 0, jnp.int32) + r
        plsc.store_scatter(buf, [rvec, rowidx], ones, mask=mask7)
        return 0

    lax.fori_loop(0, ROWS_W, row_body, 0)
    pltpu.sync_copy(buf, adj_hbm.at[b, pl.ds(r0, ROWS_W)])


@functools.partial(
    pl.kernel,
    out_type=jax.ShapeDtypeStruct((B, N, N), jnp.float32),
    mesh=plsc.VectorSubcoreMesh(core_axis_name="c", subcore_axis_name="s"),
    compiler_params=pltpu.CompilerParams(needs_layout_passes=False),
    scratch_types=[
        pltpu.VMEM((ROWS_W, 16), jnp.int32),
        pltpu.VMEM((ROWS_W, N), jnp.float32),
    ],
)
def _sc_adj(idx_hbm, adj_hbm, idx_v, buf):
    _sc_adj_body(idx_hbm, adj_hbm, idx_v, buf)


ROWS_I = (B * 3 * 384) // NW
COMBO_W = ROWS_I // 16


def _sc_im2col_body(x_hbm, xp_hbm, slab, xp_loc, sem):
    wid = lax.axis_index("s") * 2 + lax.axis_index("c")
    b = wid // 4
    ibase = (wid % 4) * 6
    for hh in range(2):
        i0 = ibase + hh * 3
        for c in range(3):
            pltpu.sync_copy(
                x_hbm.at[pl.ds((b * 3 + c) * 384 + i0 * 16, 48)],
                slab.at[pl.ds(c * 48, 48)])
        for c in range(3):

            @plsc.parallel_loop(0, 48, step=1, unroll=2)
            def move(t, c=c):
                u = t // 3
                ii = t % 3
                col = (c * 16 + u) * 16
                src_row = c * 48 + ii * 16 + u
                dst_row = ii * 24
                for j in range(24):
                    xp_loc[dst_row + j, pl.ds(col, 16)] = (
                        slab[src_row, pl.ds(j * 16, 16)])
        pltpu.sync_copy(xp_loc, xp_hbm.at[b, pl.ds(i0 * 24, 72)])


@functools.partial(
    pl.kernel,
    out_type=jax.ShapeDtypeStruct((B, N, P), jnp.float32),
    mesh=plsc.VectorSubcoreMesh(core_axis_name="c", subcore_axis_name="s"),
    compiler_params=pltpu.CompilerParams(needs_layout_passes=False),
    scratch_types=[
        pltpu.VMEM((144, 384), jnp.float32),
        pltpu.VMEM((72, P), jnp.float32),
        pltpu.SemaphoreType.DMA,
    ],
)
def _sc_im2col(x_hbm, xp_hbm, slab, xp_loc, sem):
    _sc_im2col_body(x_hbm, xp_hbm, slab, xp_loc, sem)


def kernel(x, W_conv, b_conv):
    xp = _sc_im2col(x.reshape(B * 3 * 384, 384))
    wm = W_conv.reshape(C, P).T
    bias = b_conv.reshape(1, C)
    y, idxs = pl.pallas_call(
        _tc_body,
        grid=(B,),
        in_specs=[
            pl.BlockSpec((1, N, P), lambda b: (b, 0, 0)),
            pl.BlockSpec((P, C), lambda b: (0, 0)),
            pl.BlockSpec((1, C), lambda b: (0, 0)),
        ],
        out_specs=[
            pl.BlockSpec((1, N, C), lambda b: (b, 0, 0)),
            pl.BlockSpec((1, N, 16), lambda b: (b, 0, 0)),
        ],
        out_shape=[
            jax.ShapeDtypeStruct((B, N, C), jnp.float32),
            jax.ShapeDtypeStruct((B, N, 16), jnp.int32),
        ],
    )(xp, wm, bias)
    adj = _sc_adj(idxs)
    return (y, adj)
